# baseline (speedup 1.0000x reference)
import functools

import jax
import jax.numpy as jnp
from jax import lax
from jax.experimental import pallas as pl
from jax.experimental.pallas import tpu as pltpu
from jax.experimental.pallas import tpu_sc as plsc

DIM = 64
BATCH = 16384
NC = 2
NS = 16
NW = NC * NS
CPW = BATCH // NW
CHUNK = 128
NCHUNK = CPW // CHUNK
LANES = 16
SROW = 2 * DIM


def _split_idx(src_v, samp_v, par_v):
    for i in range(CPW // LANES):
        sl = pl.ds(i * LANES, LANES)
        v = src_v[sl]
        samp_v[sl] = lax.shift_right_logical(v, 1)
        par_v[sl] = lax.shift_left(v & 1, 6)


def _body_a(human2, hm, hf,
            esm_o, esf_o,
            hv_m, hv_f, hs, hp,
            es_v, st_v,
            sem):
    wid = lax.axis_index("s") * NC + lax.axis_index("c")
    base = pl.multiple_of(wid * CPW, CPW)

    pltpu.sync_copy(hm.at[pl.ds(base, CPW)], hv_m)
    pltpu.sync_copy(hf.at[pl.ds(base, CPW)], hv_f)

    iota = lax.iota(jnp.int32, LANES)
    for (hv, out_o) in ((hv_m, esm_o), (hv_f, esf_o)):
        _split_idx(hv, hs, hp)
        for c in range(NCHUNK):
            sl = pl.ds(c * CHUNK, CHUNK)
            cp = pltpu.make_async_copy(human2.at[hs.at[sl]], es_v, sem)
            cp.start()
            cp.wait()
            for g in range(CHUNK // LANES):
                rowv = iota + (g * LANES)
                posb = rowv * DIM
                pav = hp[pl.ds(c * CHUNK + g * LANES, LANES)]

                def ebody(d, _, rowv=rowv, posb=posb, pav=pav):
                    v = plsc.load_gather(es_v, [rowv, pav + d])
                    plsc.store_scatter(st_v, [posb + d], v)
                    return 0

                lax.fori_loop(0, DIM, ebody, 0)
            pltpu.sync_copy(
                st_v, out_o.at[pl.ds((base + c * CHUNK) * DIM, CHUNK * DIM)])


def _body_b(gmf2, rm, tm, rf, tf, esm_i, esf_i,
            neg_o, dm_o, df_o,
            rv_m, tv_m, rv_f, tv_f, rs, rp, ts, tp,
            ep_v, eo_v, es_f,
            dm_v, df_v, ng_v,
            sem):
    wid = lax.axis_index("s") * NC + lax.axis_index("c")
    base = pl.multiple_of(wid * CPW, CPW)

    pltpu.sync_copy(rm.at[pl.ds(base, CPW)], rv_m)
    pltpu.sync_copy(tm.at[pl.ds(base, CPW)], tv_m)
    pltpu.sync_copy(rf.at[pl.ds(base, CPW)], rv_f)
    pltpu.sync_copy(tf.at[pl.ds(base, CPW)], tv_f)

    iota = lax.iota(jnp.int32, LANES)
    for (rv, tv, es_i, out_v) in ((rv_m, tv_m, esm_i, dm_v),
                                  (rv_f, tv_f, esf_i, df_v)):
        _split_idx(rv, rs, rp)
        _split_idx(tv, ts, tp)
        for c in range(NCHUNK):
            sl = pl.ds(c * CHUNK, CHUNK)
            cp1 = pltpu.make_async_copy(gmf2.at[rs.at[sl]], ep_v, sem)
            cp2 = pltpu.make_async_copy(gmf2.at[ts.at[sl]], eo_v, sem)
            cp3 = pltpu.make_async_copy(
                es_i.at[pl.ds((base + c * CHUNK) * DIM, CHUNK * DIM)], es_f,
                sem)
            cp1.start()
            cp2.start()
            cp3.start()
            cp1.wait()
            cp2.wait()
            cp3.wait()
            for g in range(CHUNK // LANES):
                rowv = iota + (g * LANES)
                posb = rowv * DIM
                pbv = rp[pl.ds(c * CHUNK + g * LANES, LANES)]
                pcv = tp[pl.ds(c * CHUNK + g * LANES, LANES)]

                def sbody(d, acc, rowv=rowv, posb=posb, pbv=pbv, pcv=pcv):
                    a = plsc.load_gather(es_f, [posb + d])
                    b = plsc.load_gather(ep_v, [rowv, pbv + d])
                    cc = plsc.load_gather(eo_v, [rowv, pcv + d])
                    return acc + a * b * cc

                acc = lax.fori_loop(0, DIM, sbody,
                                    jnp.zeros((LANES,), jnp.float32))
                out_v[pl.ds(c * CHUNK + g * LANES, LANES)] = acc

    for i in range(CPW // LANES):
        sl = pl.ds(i * LANES, LANES)
        ng_v[sl] = df_v[sl] - dm_v[sl]

    pltpu.sync_copy(dm_v, dm_o.at[pl.ds(base, CPW)])
    pltpu.sync_copy(df_v, df_o.at[pl.ds(base, CPW)])
    pltpu.sync_copy(ng_v, neg_o.at[pl.ds(base, CPW)])


@jax.jit
def _run(human_embeds, gmf_embeds, hm, rm, tm, hf, rf, tf):
    one = lax.optimization_barrier(jnp.float32(1.0))
    human2 = human_embeds.reshape(-1, SROW) * one
    gmf2 = gmf_embeds.reshape(-1, SROW) * one
    mesh = plsc.VectorSubcoreMesh(core_axis_name="c", subcore_axis_name="s")
    params = pltpu.CompilerParams(
        needs_layout_passes=False, use_tc_tiling_on_sc=True)
    flat = jax.ShapeDtypeStruct((BATCH * DIM,), jnp.float32)
    out = jax.ShapeDtypeStruct((BATCH,), jnp.float32)
    idx_t = pltpu.VMEM((CPW,), jnp.int32)
    row_t = pltpu.VMEM((CHUNK, SROW), jnp.float32)
    flt_t = pltpu.VMEM((CHUNK * DIM,), jnp.float32)
    res_t = pltpu.VMEM((CPW,), jnp.float32)

    ka = functools.partial(
        pl.kernel,
        out_type=[flat, flat],
        mesh=mesh,
        compiler_params=params,
        scratch_types=[
            idx_t, idx_t, idx_t, idx_t,
            row_t, flt_t,
            pltpu.SemaphoreType.DMA,
        ],
    )(_body_a)
    esm, esf = ka(human2, hm, hf)

    kb = functools.partial(
        pl.kernel,
        out_type=[out, out, out],
        mesh=mesh,
        compiler_params=params,
        scratch_types=[
            idx_t, idx_t, idx_t, idx_t, idx_t, idx_t, idx_t, idx_t,
            row_t, row_t, flt_t,
            res_t, res_t, res_t,
            pltpu.SemaphoreType.DMA,
        ],
    )(_body_b)
    return kb(gmf2, rm, tm, rf, tf, esm, esf)


def kernel(human_embeds, gmf_embeds, male_triplets, female_triplets):
    hm = male_triplets[:, 0]
    rm = male_triplets[:, 1]
    tm = male_triplets[:, 2]
    hf = female_triplets[:, 0]
    rf = female_triplets[:, 1]
    tf = female_triplets[:, 2]
    neg, dm, df = _run(human_embeds, gmf_embeds, hm, rm, tm, hf, rf, tf)
    return (neg, dm, df)


# speedup vs baseline: 1.6711x; 1.6711x over previous
import functools

import jax
import jax.numpy as jnp
from jax import lax
from jax.experimental import pallas as pl
from jax.experimental.pallas import tpu as pltpu
from jax.experimental.pallas import tpu_sc as plsc

DIM = 64
BATCH = 16384
NC = 2
NS = 16
NW = NC * NS
CPW = BATCH // NW
CHUNK = 128
NCHUNK = CPW // CHUNK
LANES = 16


def _score_chunk(es_v, ep_v, eo_v, part_f, out_v, out_base):
    iota16 = lax.iota(jnp.int32, LANES) * LANES

    def group(g, _):
        rowb = g * LANES

        def row(rr, _):
            r = rowb + rr
            v = (es_v[r, pl.ds(0, LANES)]
                 * ep_v[r, pl.ds(0, LANES)]
                 * eo_v[r, pl.ds(0, LANES)])
            for q in range(1, DIM // LANES):
                sl = pl.ds(q * LANES, LANES)
                v = v + es_v[r, sl] * ep_v[r, sl] * eo_v[r, sl]
            part_f[pl.ds(rr * LANES, LANES)] = v
            return 0

        lax.fori_loop(0, LANES, row, 0)

        def red(j, acc):
            return acc + plsc.load_gather(part_f, [iota16 + j])

        acc = lax.fori_loop(0, LANES, red, jnp.zeros((LANES,), jnp.float32))
        out_v[pl.ds(out_base + g * LANES, LANES)] = acc
        return 0

    lax.fori_loop(0, CHUNK // LANES, group, 0)


def _body(human, gmf, hm, rm, tm, hf, rf, tf,
          neg_o, dm_o, df_o,
          hmv, rmv, tmv, hfv, rfv, tfv,
          es_v, ep_v, eo_v, part_f,
          dm_v, df_v, ng_v,
          sem):
    wid = lax.axis_index("s") * NC + lax.axis_index("c")
    base = pl.multiple_of(wid * CPW, CPW)

    pltpu.sync_copy(hm.at[pl.ds(base, CPW)], hmv)
    pltpu.sync_copy(rm.at[pl.ds(base, CPW)], rmv)
    pltpu.sync_copy(tm.at[pl.ds(base, CPW)], tmv)
    pltpu.sync_copy(hf.at[pl.ds(base, CPW)], hfv)
    pltpu.sync_copy(rf.at[pl.ds(base, CPW)], rfv)
    pltpu.sync_copy(tf.at[pl.ds(base, CPW)], tfv)

    for (hv, rv, tv, out_v) in ((hmv, rmv, tmv, dm_v), (hfv, rfv, tfv, df_v)):
        for c in range(NCHUNK):
            sl = pl.ds(c * CHUNK, CHUNK)
            cp1 = pltpu.make_async_copy(human.at[hv.at[sl]], es_v, sem)
            cp2 = pltpu.make_async_copy(gmf.at[rv.at[sl]], ep_v, sem)
            cp3 = pltpu.make_async_copy(gmf.at[tv.at[sl]], eo_v, sem)
            cp1.start()
            cp2.start()
            cp3.start()
            cp1.wait()
            cp2.wait()
            cp3.wait()
            _score_chunk(es_v, ep_v, eo_v, part_f, out_v, c * CHUNK)

    for i in range(CPW // LANES):
        sl = pl.ds(i * LANES, LANES)
        ng_v[sl] = df_v[sl] - dm_v[sl]

    pltpu.sync_copy(dm_v, dm_o.at[pl.ds(base, CPW)])
    pltpu.sync_copy(df_v, df_o.at[pl.ds(base, CPW)])
    pltpu.sync_copy(ng_v, neg_o.at[pl.ds(base, CPW)])


@jax.jit
def _run(human_embeds, gmf_embeds, hm, rm, tm, hf, rf, tf):
    out = jax.ShapeDtypeStruct((BATCH,), jnp.float32)
    k = functools.partial(
        pl.kernel,
        out_type=[out, out, out],
        mesh=plsc.VectorSubcoreMesh(core_axis_name="c", subcore_axis_name="s"),
        compiler_params=pltpu.CompilerParams(
            needs_layout_passes=False, use_tc_tiling_on_sc=False),
        scratch_types=[
            pltpu.VMEM((CPW,), jnp.int32),
            pltpu.VMEM((CPW,), jnp.int32),
            pltpu.VMEM((CPW,), jnp.int32),
            pltpu.VMEM((CPW,), jnp.int32),
            pltpu.VMEM((CPW,), jnp.int32),
            pltpu.VMEM((CPW,), jnp.int32),
            pltpu.VMEM((CHUNK, DIM), jnp.float32),
            pltpu.VMEM((CHUNK, DIM), jnp.float32),
            pltpu.VMEM((CHUNK, DIM), jnp.float32),
            pltpu.VMEM((LANES * LANES,), jnp.float32),
            pltpu.VMEM((CPW,), jnp.float32),
            pltpu.VMEM((CPW,), jnp.float32),
            pltpu.VMEM((CPW,), jnp.float32),
            pltpu.SemaphoreType.DMA,
        ],
    )(_body)
    return k(human_embeds, gmf_embeds, hm, rm, tm, hf, rf, tf)


def kernel(human_embeds, gmf_embeds, male_triplets, female_triplets):
    hm = male_triplets[:, 0]
    rm = male_triplets[:, 1]
    tm = male_triplets[:, 2]
    hf = female_triplets[:, 0]
    rf = female_triplets[:, 1]
    tf = female_triplets[:, 2]
    neg, dm, df = _run(human_embeds, gmf_embeds, hm, rm, tm, hf, rf, tf)
    return (neg, dm, df)
